# baseline (speedup 1.0000x reference)
import functools

import jax
import jax.numpy as jnp
from jax import lax
from jax.experimental import pallas as pl
from jax.experimental.pallas import tpu as pltpu
from jax.experimental.pallas import tpu_sc as plsc

EMBED_DIM = 32
VOCAB_ROWS = 1000000
BATCH, SEQ = 16384, 50
B_TOTAL = BATCH * SEQ

_info = plsc.get_sparse_core_info()
_NC, _NS = _info.num_cores, _info.num_subcores
_NW = _NC * _NS
_B_PER_W = B_TOTAL // _NW
_CHUNK = 1600
_NCHUNK = _B_PER_W // _CHUNK

_mesh = plsc.VectorSubcoreMesh(core_axis_name="c", subcore_axis_name="s")


@functools.partial(
    pl.kernel,
    mesh=_mesh,
    out_type=jax.ShapeDtypeStruct((B_TOTAL, EMBED_DIM), jnp.float32),
    scratch_types=[
        pltpu.VMEM((2, _CHUNK), jnp.int32),
        pltpu.VMEM((2, _CHUNK, EMBED_DIM), jnp.float32),
        pltpu.SemaphoreType.DMA,
        pltpu.SemaphoreType.DMA,
    ],
    compiler_params=pltpu.CompilerParams(use_tc_tiling_on_sc=False),
)
def _gather(table_hbm, idx_hbm, out_hbm, idx_v, rows_v, sem0, sem1):
    wid = lax.axis_index("s") * _NC + lax.axis_index("c")
    base = wid * _B_PER_W
    sems = (sem0, sem1)

    def load_and_fire(g):
        b = g % 2
        pltpu.sync_copy(idx_hbm.at[pl.ds(base + g * _CHUNK, _CHUNK)], idx_v.at[b])
        return pltpu.async_copy(table_hbm.at[idx_v.at[b]], rows_v.at[b], sems[b])

    copies = [None] * _NCHUNK
    copies[0] = load_and_fire(0)
    for g in range(_NCHUNK):
        if g + 1 < _NCHUNK:
            copies[g + 1] = load_and_fire(g + 1)
        copies[g].wait()
        pltpu.sync_copy(rows_v.at[g % 2],
                        out_hbm.at[pl.ds(base + g * _CHUNK, _CHUNK)])


_FMT_CHUNKS = 976
_FMT_ROUNDS = 31
_TAIL_V0 = _FMT_CHUNKS * 1024
_TAIL_V = VOCAB_ROWS - _TAIL_V0
_TAIL_SUB = 192


@functools.partial(
    pl.kernel,
    mesh=_mesh,
    out_type=jax.ShapeDtypeStruct((VOCAB_ROWS // 4, 128), jnp.float32),
    scratch_types=[
        pltpu.VMEM((EMBED_DIM, 1024), jnp.float32),
        pltpu.VMEM((256, 128), jnp.float32),
        pltpu.VMEM((_TAIL_SUB, EMBED_DIM), jnp.float32),
    ],
    compiler_params=pltpu.CompilerParams(
        use_tc_tiling_on_sc=True, needs_layout_passes=False),
)
def _format(tT_hbm, tail_hbm, out_hbm, src_v, dst_v, tail_v):
    wid = lax.axis_index("s") * _NC + lax.axis_index("c")
    lane = lax.iota(jnp.int32, 16)
    for r in range(_FMT_ROUNDS):
        cid = r * 32 + wid

        @pl.when(cid < _FMT_CHUNKS)
        def _chunk():
            v0 = pl.multiple_of(cid * 1024, 1024)
            pltpu.sync_copy(tT_hbm.at[:, pl.ds(v0, 1024)], src_v)

            def row(rr, carry):
                for k in range(4):
                    vv = 4 * rr + k
                    for h in range(2):
                        vals = plsc.load_gather(
                            src_v, [h * 16 + lane, jnp.full((16,), vv, jnp.int32)])
                        dst_v[rr, pl.ds(k * 32 + h * 16, 16)] = vals
                return carry

            lax.fori_loop(0, 256, row, 0)
            pltpu.sync_copy(dst_v, out_hbm.at[pl.ds(cid * 256, 256)])

    @pl.when(wid == _NW - 1)
    def _tail():
        for t in range(_TAIL_V // _TAIL_SUB):
            pltpu.sync_copy(tail_hbm.at[pl.ds(t * _TAIL_SUB, _TAIL_SUB)], tail_v)

            def trow(rr, carry):
                for k in range(4):
                    for h in range(2):
                        dst_v[rr, pl.ds(k * 32 + h * 16, 16)] = (
                            tail_v[4 * rr + k, pl.ds(h * 16, 16)])
                return carry

            lax.fori_loop(0, _TAIL_SUB // 4, trow, 0)
            pltpu.sync_copy(
                dst_v.at[pl.ds(0, _TAIL_SUB // 4)],
                out_hbm.at[pl.ds(_TAIL_V0 // 4 + t * (_TAIL_SUB // 4),
                                 _TAIL_SUB // 4)])


def kernel(x, embed_weight):
    t2 = _format(embed_weight.T, embed_weight[_TAIL_V0:, :])
    t_lin = jnp.reshape(t2, (VOCAB_ROWS, EMBED_DIM))
    idx = x.reshape(-1).astype(jnp.int32)
    out = _gather(t_lin, idx)
    o2 = jax.lax.optimization_barrier(jnp.reshape(out, (B_TOTAL // 4, 128)))
    return jnp.reshape(o2, (BATCH, SEQ, EMBED_DIM))


# speedup vs baseline: 1.3333x; 1.3333x over previous
import functools

import jax
import jax.numpy as jnp
from jax import lax
from jax.experimental import pallas as pl
from jax.experimental.pallas import tpu as pltpu
from jax.experimental.pallas import tpu_sc as plsc

EMBED_DIM = 32
VOCAB_ROWS = 1000000
BATCH, SEQ = 16384, 50
B_TOTAL = BATCH * SEQ

_info = plsc.get_sparse_core_info()
_NC, _NS = _info.num_cores, _info.num_subcores
_NW = _NC * _NS
_B_PER_W = B_TOTAL // _NW
_CHUNK = 1600
_NCHUNK = _B_PER_W // _CHUNK

_mesh = plsc.VectorSubcoreMesh(core_axis_name="c", subcore_axis_name="s")


@functools.partial(
    pl.kernel,
    mesh=_mesh,
    out_type=jax.ShapeDtypeStruct((B_TOTAL, EMBED_DIM), jnp.float32),
    scratch_types=[
        pltpu.VMEM((2, _CHUNK), jnp.int32),
        pltpu.VMEM((2, _CHUNK, EMBED_DIM), jnp.float32),
        pltpu.SemaphoreType.DMA,
        pltpu.SemaphoreType.DMA,
    ],
    compiler_params=pltpu.CompilerParams(use_tc_tiling_on_sc=False),
)
def _gather(table_hbm, idx_hbm, out_hbm, idx_v, rows_v, sem0, sem1):
    wid = lax.axis_index("s") * _NC + lax.axis_index("c")
    base = wid * _B_PER_W
    sems = (sem0, sem1)

    def load_and_fire(g):
        b = g % 2
        pltpu.sync_copy(idx_hbm.at[pl.ds(base + g * _CHUNK, _CHUNK)], idx_v.at[b])
        return pltpu.async_copy(table_hbm.at[idx_v.at[b]], rows_v.at[b], sems[b])

    copies = [None] * _NCHUNK
    copies[0] = load_and_fire(0)
    for g in range(_NCHUNK):
        if g + 1 < _NCHUNK:
            copies[g + 1] = load_and_fire(g + 1)
        copies[g].wait()
        pltpu.sync_copy(rows_v.at[g % 2],
                        out_hbm.at[pl.ds(base + g * _CHUNK, _CHUNK)])


_FMT_CHUNKS = 976
_FMT_ROUNDS = 31
_TAIL_V0 = _FMT_CHUNKS * 1024
_TAIL_V = VOCAB_ROWS - _TAIL_V0
_TAIL_SUB = 192


@functools.partial(
    pl.kernel,
    mesh=_mesh,
    out_type=jax.ShapeDtypeStruct((VOCAB_ROWS // 4, 128), jnp.float32),
    scratch_types=[
        pltpu.VMEM((EMBED_DIM, 1024), jnp.float32),
        pltpu.VMEM((256, 128), jnp.float32),
        pltpu.VMEM((_TAIL_SUB, EMBED_DIM), jnp.float32),
    ],
    compiler_params=pltpu.CompilerParams(
        use_tc_tiling_on_sc=True, needs_layout_passes=False),
)
def _format(tT_hbm, tail_hbm, out_hbm, src_v, dst_v, tail_v):
    wid = lax.axis_index("s") * _NC + lax.axis_index("c")
    lane = lax.iota(jnp.int32, 16)
    for r in range(_FMT_ROUNDS):
        cid = r * 32 + wid

        @pl.when(cid < _FMT_CHUNKS)
        def _chunk():
            v0 = pl.multiple_of(cid * 1024, 1024)
            pltpu.sync_copy(tT_hbm.at[:, pl.ds(v0, 1024)], src_v)

            @plsc.parallel_loop(0, 256, unroll=8)
            def _row(rr):
                for k in range(4):
                    vv = 4 * rr + k
                    for h in range(2):
                        vals = plsc.load_gather(
                            src_v, [h * 16 + lane, jnp.full((16,), vv, jnp.int32)])
                        dst_v[rr, pl.ds(k * 32 + h * 16, 16)] = vals
            pltpu.sync_copy(dst_v, out_hbm.at[pl.ds(cid * 256, 256)])

    @pl.when(wid == _NW - 1)
    def _tail():
        for t in range(_TAIL_V // _TAIL_SUB):
            pltpu.sync_copy(tail_hbm.at[pl.ds(t * _TAIL_SUB, _TAIL_SUB)], tail_v)

            def trow(rr, carry):
                for k in range(4):
                    for h in range(2):
                        dst_v[rr, pl.ds(k * 32 + h * 16, 16)] = (
                            tail_v[4 * rr + k, pl.ds(h * 16, 16)])
                return carry

            lax.fori_loop(0, _TAIL_SUB // 4, trow, 0)
            pltpu.sync_copy(
                dst_v.at[pl.ds(0, _TAIL_SUB // 4)],
                out_hbm.at[pl.ds(_TAIL_V0 // 4 + t * (_TAIL_SUB // 4),
                                 _TAIL_SUB // 4)])


def kernel(x, embed_weight):
    t2 = _format(embed_weight.T, embed_weight[_TAIL_V0:, :])
    t_lin = jnp.reshape(t2, (VOCAB_ROWS, EMBED_DIM))
    idx = x.reshape(-1).astype(jnp.int32)
    out = _gather(t_lin, idx)
    o2 = jax.lax.optimization_barrier(jnp.reshape(out, (B_TOTAL // 4, 128)))
    return jnp.reshape(o2, (BATCH, SEQ, EMBED_DIM))


# speedup vs baseline: 1.3767x; 1.0326x over previous
import functools

import jax
import jax.numpy as jnp
from jax import lax
from jax.experimental import pallas as pl
from jax.experimental.pallas import tpu as pltpu
from jax.experimental.pallas import tpu_sc as plsc

EMBED_DIM = 32
VOCAB_ROWS = 1000000
BATCH, SEQ = 16384, 50
B_TOTAL = BATCH * SEQ

_info = plsc.get_sparse_core_info()
_NC, _NS = _info.num_cores, _info.num_subcores
_NW = _NC * _NS
_B_PER_W = B_TOTAL // _NW
_CHUNK = 1600
_NCHUNK = _B_PER_W // _CHUNK

_mesh = plsc.VectorSubcoreMesh(core_axis_name="c", subcore_axis_name="s")


@functools.partial(
    pl.kernel,
    mesh=_mesh,
    out_type=jax.ShapeDtypeStruct((B_TOTAL, EMBED_DIM), jnp.float32),
    scratch_types=[
        pltpu.VMEM((2, _CHUNK), jnp.int32),
        pltpu.VMEM((2, _CHUNK, EMBED_DIM), jnp.float32),
        pltpu.SemaphoreType.DMA,
        pltpu.SemaphoreType.DMA,
    ],
    compiler_params=pltpu.CompilerParams(use_tc_tiling_on_sc=False),
)
def _gather(table_hbm, idx_hbm, out_hbm, idx_v, rows_v, sem0, sem1):
    wid = lax.axis_index("s") * _NC + lax.axis_index("c")
    base = wid * _B_PER_W
    sems = (sem0, sem1)

    def load_and_fire(g):
        b = g % 2
        pltpu.sync_copy(idx_hbm.at[pl.ds(base + g * _CHUNK, _CHUNK)], idx_v.at[b])
        return pltpu.async_copy(table_hbm.at[idx_v.at[b]], rows_v.at[b], sems[b])

    copies = [None] * _NCHUNK
    copies[0] = load_and_fire(0)
    for g in range(_NCHUNK):
        if g + 1 < _NCHUNK:
            copies[g + 1] = load_and_fire(g + 1)
        copies[g].wait()
        pltpu.sync_copy(rows_v.at[g % 2],
                        out_hbm.at[pl.ds(base + g * _CHUNK, _CHUNK)])


def kernel(x, embed_weight):
    t2 = jax.lax.optimization_barrier(jnp.reshape(embed_weight, (250000, 128)))
    t_lin = jnp.reshape(t2, (VOCAB_ROWS, EMBED_DIM))
    idx = x.reshape(-1).astype(jnp.int32)
    out = _gather(t_lin, idx)
    o2 = jax.lax.optimization_barrier(jnp.reshape(out, (B_TOTAL // 4, 128)))
    return jnp.reshape(o2, (BATCH, SEQ, EMBED_DIM))
